# Initial kernel scaffold; baseline (speedup 1.0000x reference)
import jax
import jax.numpy as jnp
from jax.experimental import pallas as pl


def kernel(times, values, t):
    raise NotImplementedError("write your pallas kernel here")


# speedup vs baseline: 9.7798x; 9.7798x over previous
import functools

import jax
import jax.numpy as jnp
from jax import lax
from jax.experimental import pallas as pl
from jax.experimental.pallas import tpu as pltpu
from jax.experimental.pallas import tpu_sc as plsc

NTIME, NBATCH, K = 100, 1024, 128
STRIDE = 128
NC, NS = 2, 16
NW = NC * NS
CW = NBATCH // NW
WORDS = CW * STRIDE


def _interp_body(times_hbm, values_hbm, t_hbm, out_hbm,
                 times_v, values_v, slopes_v, t_v, out_v):
    wid = lax.axis_index("s") * NC + lax.axis_index("c")
    base0 = wid * WORDS
    pltpu.sync_copy(times_hbm.at[pl.ds(base0, WORDS)], times_v)
    pltpu.sync_copy(values_hbm.at[pl.ds(base0, WORDS)], values_v)
    pltpu.sync_copy(t_hbm.at[pl.ds(base0, WORDS)], t_v)

    def col_body(c, carry):
        base = c * STRIDE
        for ci in (0, 16, 32, 48, 64, 80, 83):
            v0 = values_v[pl.ds(base + ci, 16)]
            v1 = values_v[pl.ds(base + ci + 1, 16)]
            t0 = times_v[pl.ds(base + ci, 16)]
            t1 = times_v[pl.ds(base + ci + 1, 16)]
            slopes_v[pl.ds(base + ci, 16)] = (v1 - v0) / (t1 - t0)
        for q in range(8):
            tq = t_v[pl.ds(base + q * 16, 16)]
            pos = jnp.zeros((16,), jnp.int32)
            for w in (64, 32, 16, 8, 4, 2, 1):
                g = plsc.load_gather(times_v, [base + (pos + (w - 1))])
                pos = jnp.where(g <= tq, pos + w, pos)
            is0 = (pos == 0) | (pos == NTIME)
            iv = jnp.where(is0, NTIME - 1, pos - 1)
            isl = jnp.where(is0, NTIME - 2, pos - 1)
            gv = plsc.load_gather(values_v, [base + iv])
            gt = plsc.load_gather(times_v, [base + iv])
            gs = plsc.load_gather(slopes_v, [base + isl])
            out_v[pl.ds(base + q * 16, 16)] = gv + gs * (tq - gt)
        return carry

    lax.fori_loop(0, CW, col_body, 0)
    pltpu.sync_copy(out_v, out_hbm.at[pl.ds(base0, WORDS)])


@jax.jit
def _run(times_p, values_p, t_p):
    mesh = plsc.VectorSubcoreMesh(core_axis_name="c", subcore_axis_name="s")
    f = functools.partial(
        pl.kernel,
        mesh=mesh,
        compiler_params=pltpu.CompilerParams(needs_layout_passes=False),
        out_type=jax.ShapeDtypeStruct((NBATCH * K,), jnp.float32),
        scratch_types=[
            pltpu.VMEM((WORDS,), jnp.float32),
            pltpu.VMEM((WORDS,), jnp.float32),
            pltpu.VMEM((WORDS,), jnp.float32),
            pltpu.VMEM((WORDS,), jnp.float32),
            pltpu.VMEM((WORDS,), jnp.float32),
        ],
    )(_interp_body)
    return f(times_p, values_p, t_p)


def kernel(times, values, t):
    pad = jnp.full((NBATCH, STRIDE - NTIME), jnp.inf, jnp.float32)
    times_p = jnp.concatenate([times.T, pad], axis=1).reshape(-1)
    values_p = jnp.concatenate([values.T, jnp.zeros_like(pad)], axis=1).reshape(-1)
    t_p = t.T.reshape(-1)
    out = _run(times_p, values_p, t_p)
    return out.reshape(NBATCH, K).T


# speedup vs baseline: 11.4940x; 1.1753x over previous
import functools

import jax
import jax.numpy as jnp
from jax import lax
from jax.experimental import pallas as pl
from jax.experimental.pallas import tpu as pltpu
from jax.experimental.pallas import tpu_sc as plsc

NTIME, NBATCH, K = 100, 1024, 128
NPAD = 128
NC, NS = 2, 16
NW = NC * NS
CW = NBATCH // NW
LANES = 16


def _interp_body(times_hbm, values_hbm, t_hbm, out_hbm,
                 times_v, values_v, slopes_v, t_v, out_v):
    wid = lax.axis_index("s") * NC + lax.axis_index("c")
    c0 = wid * CW
    pltpu.sync_copy(times_hbm.at[:, pl.ds(c0, CW)], times_v.at[pl.ds(0, NTIME), :])
    pltpu.sync_copy(values_hbm.at[:, pl.ds(c0, CW)], values_v)
    pltpu.sync_copy(t_hbm.at[:, pl.ds(c0, CW)], t_v)

    inf16 = jnp.full((LANES,), jnp.inf, jnp.float32)
    for r in range(NTIME, NPAD):
        for g in range(CW // LANES):
            times_v[r, pl.ds(g * LANES, LANES)] = inf16

    def slope_body(i, carry):
        for g in range(CW // LANES):
            v0 = values_v[i, pl.ds(g * LANES, LANES)]
            v1 = values_v[i + 1, pl.ds(g * LANES, LANES)]
            t0 = times_v[i, pl.ds(g * LANES, LANES)]
            t1 = times_v[i + 1, pl.ds(g * LANES, LANES)]
            slopes_v[i, pl.ds(g * LANES, LANES)] = (v1 - v0) / (t1 - t0)
        return carry

    lax.fori_loop(0, NTIME - 1, slope_body, 0, unroll=4)

    def query_body(k, carry):
        for g in range(CW // LANES):
            coff = lax.iota(jnp.int32, LANES) + (g * LANES)
            tq = t_v[k, pl.ds(g * LANES, LANES)]
            pos = jnp.zeros((LANES,), jnp.int32)
            for w in (64, 32, 16, 8, 4, 2, 1):
                gk = plsc.load_gather(times_v, [pos + (w - 1), coff])
                pos = jnp.where(gk <= tq, pos + w, pos)
            is0 = (pos == 0) | (pos == NTIME)
            iv = jnp.where(is0, NTIME - 1, pos - 1)
            isl = jnp.where(is0, NTIME - 2, pos - 1)
            gv = plsc.load_gather(values_v, [iv, coff])
            gt = plsc.load_gather(times_v, [iv, coff])
            gs = plsc.load_gather(slopes_v, [isl, coff])
            out_v[k, pl.ds(g * LANES, LANES)] = gv + gs * (tq - gt)
        return carry

    lax.fori_loop(0, K, query_body, 0, unroll=2)
    pltpu.sync_copy(out_v, out_hbm.at[:, pl.ds(c0, CW)])


@jax.jit
def _run(times, values, t):
    mesh = plsc.VectorSubcoreMesh(core_axis_name="c", subcore_axis_name="s")
    f = functools.partial(
        pl.kernel,
        mesh=mesh,
        compiler_params=pltpu.CompilerParams(
            needs_layout_passes=False, use_tc_tiling_on_sc=False),
        out_type=jax.ShapeDtypeStruct((K, NBATCH), jnp.float32),
        scratch_types=[
            pltpu.VMEM((NPAD, CW), jnp.float32),
            pltpu.VMEM((NTIME, CW), jnp.float32),
            pltpu.VMEM((NTIME - 1, CW), jnp.float32),
            pltpu.VMEM((K, CW), jnp.float32),
            pltpu.VMEM((K, CW), jnp.float32),
        ],
    )(_interp_body)
    return f(times, values, t)


def kernel(times, values, t):
    return _run(times, values, t)


# speedup vs baseline: 12.4915x; 1.0868x over previous
import functools

import jax
import jax.numpy as jnp
from jax import lax
from jax.experimental import pallas as pl
from jax.experimental.pallas import tpu as pltpu
from jax.experimental.pallas import tpu_sc as plsc

NTIME, NBATCH, K = 100, 1024, 128
NPAD = 128
NC, NS = 2, 16
NW = NC * NS
CW = NBATCH // NW
LANES = 16


def _interp_body(times_hbm, values_hbm, t_hbm, out_hbm,
                 times_v, values_v, slopes_v, t_v, out_v):
    wid = lax.axis_index("s") * NC + lax.axis_index("c")
    c0 = wid * CW
    pltpu.sync_copy(times_hbm.at[:, pl.ds(c0, CW)], times_v.at[pl.ds(0, NTIME), :])
    pltpu.sync_copy(values_hbm.at[:, pl.ds(c0, CW)], values_v)
    pltpu.sync_copy(t_hbm.at[:, pl.ds(c0, CW)], t_v)

    inf16 = jnp.full((LANES,), jnp.inf, jnp.float32)
    for r in range(NTIME, NPAD):
        for g in range(CW // LANES):
            times_v[r, pl.ds(g * LANES, LANES)] = inf16

    def slope_body(i, carry):
        for g in range(CW // LANES):
            v0 = values_v[i, pl.ds(g * LANES, LANES)]
            v1 = values_v[i + 1, pl.ds(g * LANES, LANES)]
            t0 = times_v[i, pl.ds(g * LANES, LANES)]
            t1 = times_v[i + 1, pl.ds(g * LANES, LANES)]
            slopes_v[i, pl.ds(g * LANES, LANES)] = (v1 - v0) / (t1 - t0)
        return carry

    lax.fori_loop(0, NTIME - 1, slope_body, 0, unroll=4)

    NG = CW // LANES
    pre = []
    for g in range(NG):
        sl = pl.ds(g * LANES, LANES)
        pre.append(tuple(times_v[r, sl] for r in (63, 31, 95, 15, 47, 79, 111)))

    def query_body(k, carry):
        for g in range(NG):
            coff = lax.iota(jnp.int32, LANES) + (g * LANES)
            t63, t31, t95, t15, t47, t79, t111 = pre[g]
            tq = t_v[k, pl.ds(g * LANES, LANES)]
            b64 = t63 <= tq
            pos = jnp.where(b64, 64, 0)
            b32 = jnp.where(b64, t95, t31) <= tq
            pos = jnp.where(b32, pos + 32, pos)
            m3 = jnp.where(b32,
                           jnp.where(b64, t111, t47),
                           jnp.where(b64, t79, t15))
            pos = jnp.where(m3 <= tq, pos + 16, pos)
            for w in (8, 4, 2, 1):
                gk = plsc.load_gather(times_v, [pos + (w - 1), coff])
                pos = jnp.where(gk <= tq, pos + w, pos)
            is0 = (pos == 0) | (pos == NTIME)
            iv = jnp.where(is0, NTIME - 1, pos - 1)
            isl = jnp.where(is0, NTIME - 2, pos - 1)
            gv = plsc.load_gather(values_v, [iv, coff])
            gt = plsc.load_gather(times_v, [iv, coff])
            gs = plsc.load_gather(slopes_v, [isl, coff])
            out_v[k, pl.ds(g * LANES, LANES)] = gv + gs * (tq - gt)
        return carry

    lax.fori_loop(0, K, query_body, 0, unroll=2)
    pltpu.sync_copy(out_v, out_hbm.at[:, pl.ds(c0, CW)])


@jax.jit
def _run(times, values, t):
    mesh = plsc.VectorSubcoreMesh(core_axis_name="c", subcore_axis_name="s")
    f = functools.partial(
        pl.kernel,
        mesh=mesh,
        compiler_params=pltpu.CompilerParams(
            needs_layout_passes=False, use_tc_tiling_on_sc=False),
        out_type=jax.ShapeDtypeStruct((K, NBATCH), jnp.float32),
        scratch_types=[
            pltpu.VMEM((NPAD, CW), jnp.float32),
            pltpu.VMEM((NTIME, CW), jnp.float32),
            pltpu.VMEM((NTIME - 1, CW), jnp.float32),
            pltpu.VMEM((K, CW), jnp.float32),
            pltpu.VMEM((K, CW), jnp.float32),
        ],
    )(_interp_body)
    return f(times, values, t)


def kernel(times, values, t):
    return _run(times, values, t)
